# Initial kernel scaffold; baseline (speedup 1.0000x reference)
import functools

import jax
import jax.numpy as jnp
from jax import lax
from jax.experimental import pallas as pl
from jax.experimental.pallas import tpu as pltpu
from jax.experimental.pallas import tpu_sc as plsc

N = 10000
E = 320000
D_IN = 128
D_H = 256
D_OUT = 128
DHALF = D_H // 2

NS = 16
NC = 2
B = 128
NB = 157
E_PAD = NS * NB * B
ACC_N = 10240
ROWS_PER_TILE = ACC_N // NS

_mesh = plsc.VectorSubcoreMesh(core_axis_name="c", subcore_axis_name="s")


def _deg_body(dst3, ones8, zeros8, out, dst2d, ones_v, acc):
    c = lax.axis_index("c")
    sid = lax.axis_index("s")

    @pl.when(c == 0)
    def _():
        pltpu.sync_copy(zeros8, acc.at[pl.ds(sid * ROWS_PER_TILE, ROWS_PER_TILE)])
        pltpu.sync_copy(dst3.at[sid], dst2d)
        pltpu.sync_copy(ones8, ones_v)
        plsc.subcore_barrier()

        def body(j, carry):
            pltpu.sync_copy(ones_v, acc.at[dst2d.at[j]], add=True)
            return carry

        lax.fori_loop(0, NB, body, 0)
        plsc.subcore_barrier()
        sl = pl.ds(sid * ROWS_PER_TILE, ROWS_PER_TILE)
        pltpu.sync_copy(acc.at[sl], out.at[sl])


_deg_kernel = pl.kernel(
    _deg_body,
    out_type=jax.ShapeDtypeStruct((ACC_N, 8), jnp.float32),
    mesh=_mesh,
    scratch_types=[
        pltpu.VMEM((NB, B), jnp.int32),
        pltpu.VMEM((B, 8), jnp.float32),
        pltpu.VMEM_SHARED((ACC_N, 8), jnp.float32),
    ],
)


def _agg_body(src3, dst3, hlo, hhi, zeros, out, src2d, dst2d, rows, acc, sem):
    c = lax.axis_index("c")
    sid = lax.axis_index("s")
    sl = pl.ds(sid * ROWS_PER_TILE, ROWS_PER_TILE)
    pltpu.sync_copy(zeros, acc.at[sl])
    pltpu.sync_copy(src3.at[sid], src2d)
    pltpu.sync_copy(dst3.at[sid], dst2d)
    plsc.subcore_barrier()

    def run(h_ref):
        def body(j, carry):
            pltpu.async_copy(h_ref.at[src2d.at[j]], rows, sem).wait()
            pltpu.sync_copy(rows, acc.at[dst2d.at[j]], add=True)
            return carry

        lax.fori_loop(0, NB, body, 0)

    pl.when(c == 0)(lambda: run(hlo))
    pl.when(c == 1)(lambda: run(hhi))
    plsc.subcore_barrier()
    pltpu.sync_copy(acc.at[sl], out.at[c, sl])


_agg_kernel = pl.kernel(
    _agg_body,
    out_type=jax.ShapeDtypeStruct((NC, ACC_N, DHALF), jnp.float32),
    mesh=_mesh,
    scratch_types=[
        pltpu.VMEM((NB, B), jnp.int32),
        pltpu.VMEM((NB, B), jnp.int32),
        pltpu.VMEM((B, DHALF), jnp.float32),
        pltpu.VMEM_SHARED((ACC_N, DHALF), jnp.float32),
        pltpu.SemaphoreType.DMA,
    ],
)


RB = 500
GRID = N // RB


def _tc0_body(x_ref, w_ref, degp_ref, hlo_ref, hhi_ref):
    dinv = lax.rsqrt(1.0 + degp_ref[:, 0:1])
    h = jnp.dot(x_ref[...], w_ref[...], preferred_element_type=jnp.float32) * dinv
    hlo_ref[...] = h[:, :DHALF]
    hhi_ref[...] = h[:, DHALF:]


def _tcmid_body(s_ref, hlo_ref, hhi_ref, degp_ref, b_ref, w_ref, olo_ref, ohi_ref):
    dinv = lax.rsqrt(1.0 + degp_ref[:, 0:1])
    za = jnp.maximum(dinv * (s_ref[0] + hlo_ref[...]) + b_ref[0:1, :DHALF], 0.0)
    zb = jnp.maximum(dinv * (s_ref[1] + hhi_ref[...]) + b_ref[0:1, DHALF:], 0.0)
    z = jnp.concatenate([za, zb], axis=1)
    h = jnp.dot(z, w_ref[...], preferred_element_type=jnp.float32) * dinv
    olo_ref[...] = h[:, :DHALF]
    ohi_ref[...] = h[:, DHALF:]


def _tchead_body(s_ref, hlo_ref, hhi_ref, degp_ref, b_ref, wl_ref, bl_ref, out_ref):
    dinv = lax.rsqrt(1.0 + degp_ref[:, 0:1])
    za = jnp.maximum(dinv * (s_ref[0] + hlo_ref[...]) + b_ref[0:1, :DHALF], 0.0)
    zb = jnp.maximum(dinv * (s_ref[1] + hhi_ref[...]) + b_ref[0:1, DHALF:], 0.0)
    z = jnp.concatenate([za, zb], axis=1)
    out_ref[...] = (
        jnp.dot(z, wl_ref[...], preferred_element_type=jnp.float32) + bl_ref[0:1, :]
    )


def _row_spec(d):
    return pl.BlockSpec((RB, d), lambda i: (i, 0))


_SPEC_S = pl.BlockSpec((NC, RB, DHALF), lambda i: (0, i, 0))
_SPEC_DEG = pl.BlockSpec((RB, 8), lambda i: (i, 0))


def _tc0(x, w0, degp):
    return pl.pallas_call(
        _tc0_body,
        grid=(GRID,),
        in_specs=[
            _row_spec(D_IN),
            pl.BlockSpec((D_IN, D_H), lambda i: (0, 0)),
            _SPEC_DEG,
        ],
        out_specs=[_row_spec(DHALF), _row_spec(DHALF)],
        out_shape=[
            jax.ShapeDtypeStruct((N, DHALF), jnp.float32),
            jax.ShapeDtypeStruct((N, DHALF), jnp.float32),
        ],
    )(x, w0, degp)


def _tcmid(s, hlo, hhi, degp, b2d, w):
    return pl.pallas_call(
        _tcmid_body,
        grid=(GRID,),
        in_specs=[
            _SPEC_S,
            _row_spec(DHALF),
            _row_spec(DHALF),
            _SPEC_DEG,
            pl.BlockSpec((1, D_H), lambda i: (0, 0)),
            pl.BlockSpec((D_H, D_H), lambda i: (0, 0)),
        ],
        out_specs=[_row_spec(DHALF), _row_spec(DHALF)],
        out_shape=[
            jax.ShapeDtypeStruct((N, DHALF), jnp.float32),
            jax.ShapeDtypeStruct((N, DHALF), jnp.float32),
        ],
    )(s, hlo, hhi, degp, b2d, w)


def _tchead(s, hlo, hhi, degp, b2d, wl, bl2d):
    return pl.pallas_call(
        _tchead_body,
        grid=(GRID,),
        in_specs=[
            _SPEC_S,
            _row_spec(DHALF),
            _row_spec(DHALF),
            _SPEC_DEG,
            pl.BlockSpec((1, D_H), lambda i: (0, 0)),
            pl.BlockSpec((D_H, D_OUT), lambda i: (0, 0)),
            pl.BlockSpec((1, D_OUT), lambda i: (0, 0)),
        ],
        out_specs=_row_spec(D_OUT),
        out_shape=jax.ShapeDtypeStruct((N, D_OUT), jnp.float32),
    )(s, hlo, hhi, degp, b2d, wl, bl2d)


def kernel(x, edge_index, W0, b0, W1, b1, W2, b2, Wl, bl):
    src = edge_index[0].astype(jnp.int32)
    dst = edge_index[1].astype(jnp.int32)
    pad = E_PAD - E
    src3 = jnp.concatenate([src, jnp.zeros((pad,), jnp.int32)]).reshape(NS, NB, B)
    dst3 = jnp.concatenate([dst, jnp.full((pad,), N, jnp.int32)]).reshape(NS, NB, B)

    zeros128 = jnp.zeros((ROWS_PER_TILE, DHALF), jnp.float32)
    zeros8 = jnp.zeros((ROWS_PER_TILE, 8), jnp.float32)
    ones8 = jnp.ones((B, 8), jnp.float32)

    degp = _deg_kernel(dst3, ones8, zeros8)

    hlo, hhi = _tc0(x, W0, degp)
    s = _agg_kernel(src3, dst3, hlo, hhi, zeros128)
    hlo, hhi = _tcmid(s, hlo, hhi, degp, b0.reshape(1, D_H), W1)
    s = _agg_kernel(src3, dst3, hlo, hhi, zeros128)
    hlo, hhi = _tcmid(s, hlo, hhi, degp, b1.reshape(1, D_H), W2)
    s = _agg_kernel(src3, dst3, hlo, hhi, zeros128)
    return _tchead(s, hlo, hhi, degp, b2.reshape(1, D_H), Wl, bl.reshape(1, D_OUT))


# speedup vs baseline: 6.3450x; 6.3450x over previous
import functools

import jax
import jax.numpy as jnp
from jax import lax
from jax.experimental import pallas as pl
from jax.experimental.pallas import tpu as pltpu
from jax.experimental.pallas import tpu_sc as plsc

N = 10000
E = 320000
D_IN = 128
D_H = 256
D_OUT = 128
DHALF = D_H // 2

NS = 16
NC = 2
B = 128
CH = 16
NCH = 10
NB = CH * NCH
E_PAD = NS * NB * B
ACC_N = 10240
ROWS_PER_TILE = ACC_N // NS

_mesh = plsc.VectorSubcoreMesh(core_axis_name="c", subcore_axis_name="s")


def _deg_body(dst4, ones128, zeros, out, didx, ones_v, acc):
    c = lax.axis_index("c")
    sid = lax.axis_index("s")
    sl = pl.ds(sid * ROWS_PER_TILE, ROWS_PER_TILE)
    pltpu.sync_copy(zeros, acc.at[sl])
    pltpu.sync_copy(ones128, ones_v)
    plsc.subcore_barrier()

    def chunk(ch, carry):
        pltpu.sync_copy(dst4.at[sid, ch], didx)

        def body(j, carry2):
            pltpu.sync_copy(ones_v, acc.at[didx.at[j]], add=True)
            return carry2

        return lax.fori_loop(0, CH, body, carry)

    half = NCH // 2
    lax.fori_loop(c * half, (c + 1) * half, chunk, 0)
    plsc.subcore_barrier()
    pltpu.sync_copy(acc.at[sl], out.at[c, sl])


_deg_kernel = pl.kernel(
    _deg_body,
    out_type=jax.ShapeDtypeStruct((NC, ACC_N, DHALF), jnp.float32),
    mesh=_mesh,
    scratch_types=[
        pltpu.VMEM((CH, B), jnp.int32),
        pltpu.VMEM((B, DHALF), jnp.float32),
        pltpu.VMEM_SHARED((ACC_N, DHALF), jnp.float32),
    ],
)


def _agg_body(src4, dst4, hlo, hhi, zeros, out, sidx, didx, rows, acc, sem):
    c = lax.axis_index("c")
    sid = lax.axis_index("s")
    sl = pl.ds(sid * ROWS_PER_TILE, ROWS_PER_TILE)
    pltpu.sync_copy(zeros, acc.at[sl])
    plsc.subcore_barrier()

    def run(h_ref):
        def chunk(ch, carry):
            pltpu.sync_copy(src4.at[sid, ch], sidx)
            pltpu.sync_copy(dst4.at[sid, ch], didx)

            def body(j, carry2):
                pltpu.async_copy(h_ref.at[sidx.at[j]], rows, sem).wait()
                pltpu.sync_copy(rows, acc.at[didx.at[j]], add=True)
                return carry2

            return lax.fori_loop(0, CH, body, carry)

        lax.fori_loop(0, NCH, chunk, 0)

    pl.when(c == 0)(lambda: run(hlo))
    pl.when(c == 1)(lambda: run(hhi))
    plsc.subcore_barrier()
    pltpu.sync_copy(acc.at[sl], out.at[c, sl])


_agg_kernel = pl.kernel(
    _agg_body,
    out_type=jax.ShapeDtypeStruct((NC, ACC_N, DHALF), jnp.float32),
    mesh=_mesh,
    scratch_types=[
        pltpu.VMEM((CH, B), jnp.int32),
        pltpu.VMEM((CH, B), jnp.int32),
        pltpu.VMEM((B, DHALF), jnp.float32),
        pltpu.VMEM_SHARED((ACC_N, DHALF), jnp.float32),
        pltpu.SemaphoreType.DMA,
    ],
)


RB = 1000
GRID = N // RB


def _tc0_body(x_ref, w_ref, degp_ref, hlo_ref, hhi_ref):
    dinv = lax.rsqrt(1.0 + degp_ref[0, :, 0:1] + degp_ref[1, :, 0:1])
    h = jnp.dot(x_ref[...], w_ref[...], preferred_element_type=jnp.float32) * dinv
    hlo_ref[...] = h[:, :DHALF]
    hhi_ref[...] = h[:, DHALF:]


def _tcmid_body(s_ref, hlo_ref, hhi_ref, degp_ref, b_ref, w_ref, olo_ref, ohi_ref):
    dinv = lax.rsqrt(1.0 + degp_ref[0, :, 0:1] + degp_ref[1, :, 0:1])
    za = jnp.maximum(dinv * (s_ref[0] + hlo_ref[...]) + b_ref[0:1, :DHALF], 0.0)
    zb = jnp.maximum(dinv * (s_ref[1] + hhi_ref[...]) + b_ref[0:1, DHALF:], 0.0)
    z = jnp.concatenate([za, zb], axis=1)
    h = jnp.dot(z, w_ref[...], preferred_element_type=jnp.float32) * dinv
    olo_ref[...] = h[:, :DHALF]
    ohi_ref[...] = h[:, DHALF:]


def _tchead_body(s_ref, hlo_ref, hhi_ref, degp_ref, b_ref, wl_ref, bl_ref, out_ref):
    dinv = lax.rsqrt(1.0 + degp_ref[0, :, 0:1] + degp_ref[1, :, 0:1])
    za = jnp.maximum(dinv * (s_ref[0] + hlo_ref[...]) + b_ref[0:1, :DHALF], 0.0)
    zb = jnp.maximum(dinv * (s_ref[1] + hhi_ref[...]) + b_ref[0:1, DHALF:], 0.0)
    z = jnp.concatenate([za, zb], axis=1)
    out_ref[...] = (
        jnp.dot(z, wl_ref[...], preferred_element_type=jnp.float32) + bl_ref[0:1, :]
    )


def _row_spec(d):
    return pl.BlockSpec((RB, d), lambda i: (i, 0))


_SPEC_S = pl.BlockSpec((NC, RB, DHALF), lambda i: (0, i, 0))
_SPEC_DEG = pl.BlockSpec((NC, RB, DHALF), lambda i: (0, i, 0))


def _tc0(x, w0, degp):
    return pl.pallas_call(
        _tc0_body,
        grid=(GRID,),
        in_specs=[
            _row_spec(D_IN),
            pl.BlockSpec((D_IN, D_H), lambda i: (0, 0)),
            _SPEC_DEG,
        ],
        out_specs=[_row_spec(DHALF), _row_spec(DHALF)],
        out_shape=[
            jax.ShapeDtypeStruct((N, DHALF), jnp.float32),
            jax.ShapeDtypeStruct((N, DHALF), jnp.float32),
        ],
    )(x, w0, degp)


def _tcmid(s, hlo, hhi, degp, b2d, w):
    return pl.pallas_call(
        _tcmid_body,
        grid=(GRID,),
        in_specs=[
            _SPEC_S,
            _row_spec(DHALF),
            _row_spec(DHALF),
            _SPEC_DEG,
            pl.BlockSpec((1, D_H), lambda i: (0, 0)),
            pl.BlockSpec((D_H, D_H), lambda i: (0, 0)),
        ],
        out_specs=[_row_spec(DHALF), _row_spec(DHALF)],
        out_shape=[
            jax.ShapeDtypeStruct((N, DHALF), jnp.float32),
            jax.ShapeDtypeStruct((N, DHALF), jnp.float32),
        ],
    )(s, hlo, hhi, degp, b2d, w)


def _tchead(s, hlo, hhi, degp, b2d, wl, bl2d):
    return pl.pallas_call(
        _tchead_body,
        grid=(GRID,),
        in_specs=[
            _SPEC_S,
            _row_spec(DHALF),
            _row_spec(DHALF),
            _SPEC_DEG,
            pl.BlockSpec((1, D_H), lambda i: (0, 0)),
            pl.BlockSpec((D_H, D_OUT), lambda i: (0, 0)),
            pl.BlockSpec((1, D_OUT), lambda i: (0, 0)),
        ],
        out_specs=_row_spec(D_OUT),
        out_shape=jax.ShapeDtypeStruct((N, D_OUT), jnp.float32),
    )(s, hlo, hhi, degp, b2d, wl, bl2d)


def kernel(x, edge_index, W0, b0, W1, b1, W2, b2, Wl, bl):
    src = edge_index[0].astype(jnp.int32)
    dst = edge_index[1].astype(jnp.int32)
    pad = E_PAD - E
    src3 = jnp.concatenate([src, jnp.zeros((pad,), jnp.int32)]).reshape(
        NS, NCH, CH, B
    )
    dst3 = jnp.concatenate([dst, jnp.full((pad,), N, jnp.int32)]).reshape(
        NS, NCH, CH, B
    )

    zeros128 = jnp.zeros((ROWS_PER_TILE, DHALF), jnp.float32)
    ones128 = jnp.ones((B, DHALF), jnp.float32)

    degp = _deg_kernel(dst3, ones128, zeros128)

    hlo, hhi = _tc0(x, W0, degp)
    s = _agg_kernel(src3, dst3, hlo, hhi, zeros128)
    hlo, hhi = _tcmid(s, hlo, hhi, degp, b0.reshape(1, D_H), W1)
    s = _agg_kernel(src3, dst3, hlo, hhi, zeros128)
    hlo, hhi = _tcmid(s, hlo, hhi, degp, b1.reshape(1, D_H), W2)
    s = _agg_kernel(src3, dst3, hlo, hhi, zeros128)
    return _tchead(s, hlo, hhi, degp, b2.reshape(1, D_H), Wl, bl.reshape(1, D_OUT))


# speedup vs baseline: 7.0641x; 1.1133x over previous
import functools

import jax
import jax.numpy as jnp
from jax import lax
from jax.experimental import pallas as pl
from jax.experimental.pallas import tpu as pltpu
from jax.experimental.pallas import tpu_sc as plsc

N = 10000
E = 320000
D_IN = 128
D_H = 256
D_OUT = 128
DHALF = D_H // 2

NS = 16
NC = 2
B = 128
CH = 16
NCH = 10
NB = CH * NCH
E_PAD = NS * NB * B
ACC_N = 10240
ROWS_PER_TILE = ACC_N // NS

_mesh = plsc.VectorSubcoreMesh(core_axis_name="c", subcore_axis_name="s")


def _deg_body(dst4, ones128, zeros, out, didx, ones_v, acc, semd):
    c = lax.axis_index("c")
    sid = lax.axis_index("s")
    sl = pl.ds(sid * ROWS_PER_TILE, ROWS_PER_TILE)
    pltpu.sync_copy(zeros, acc.at[sl])
    pltpu.sync_copy(ones128, ones_v)
    plsc.subcore_barrier()

    def chunk(ch, carry):
        pltpu.sync_copy(dst4.at[sid, ch], didx)

        def fire(j, carry2):
            pltpu.async_copy(ones_v, acc.at[didx.at[j]], semd, add=True)
            return carry2

        lax.fori_loop(0, CH, fire, 0)

        def drain(j, carry2):
            pltpu.make_async_copy(ones_v, acc.at[didx.at[0]], semd).wait()
            return carry2

        return lax.fori_loop(0, CH, drain, carry)

    half = NCH // 2
    lax.fori_loop(c * half, (c + 1) * half, chunk, 0)
    plsc.subcore_barrier()
    pltpu.sync_copy(acc.at[sl], out.at[c, sl])


_deg_kernel = pl.kernel(
    _deg_body,
    out_type=jax.ShapeDtypeStruct((NC, ACC_N, DHALF), jnp.float32),
    mesh=_mesh,
    scratch_types=[
        pltpu.VMEM((CH, B), jnp.int32),
        pltpu.VMEM((B, DHALF), jnp.float32),
        pltpu.VMEM_SHARED((ACC_N, DHALF), jnp.float32),
        pltpu.SemaphoreType.DMA,
    ],
)


def _agg_body(
    src4, dst4, hlo, hhi, zeros, out,
    sidx, didx, rows0, rows1, acc, semg0, semg1, sems0, sems1,
):
    c = lax.axis_index("c")
    sid = lax.axis_index("s")
    sl = pl.ds(sid * ROWS_PER_TILE, ROWS_PER_TILE)
    pltpu.sync_copy(zeros, acc.at[sl])
    plsc.subcore_barrier()

    def run(h_ref):
        def g_wait(buf, semg):
            pltpu.make_async_copy(h_ref.at[sidx.at[0]], buf, semg).wait()

        def s_wait(buf, sems):
            pltpu.make_async_copy(buf, acc.at[didx.at[0]], sems).wait()

        def chunk(ch, carry):
            pltpu.sync_copy(src4.at[sid, ch], sidx)
            pltpu.sync_copy(dst4.at[sid, ch], didx)
            pltpu.async_copy(h_ref.at[sidx.at[0]], rows0, semg0)

            def pair(k, carry2):
                j0 = 2 * k
                j1 = j0 + 1
                g_wait(rows0, semg0)
                pltpu.async_copy(rows0, acc.at[didx.at[j0]], sems0, add=True)
                pl.when(k > 0)(lambda: s_wait(rows1, sems1))
                pltpu.async_copy(h_ref.at[sidx.at[j1]], rows1, semg1)
                g_wait(rows1, semg1)
                pltpu.async_copy(rows1, acc.at[didx.at[j1]], sems1, add=True)
                s_wait(rows0, sems0)

                @pl.when(k < CH // 2 - 1)
                def _():
                    pltpu.async_copy(h_ref.at[sidx.at[j0 + 2]], rows0, semg0)

                return carry2

            out_c = lax.fori_loop(0, CH // 2, pair, carry)
            s_wait(rows1, sems1)
            return out_c

        lax.fori_loop(0, NCH, chunk, 0)

    pl.when(c == 0)(lambda: run(hlo))
    pl.when(c == 1)(lambda: run(hhi))
    plsc.subcore_barrier()
    pltpu.sync_copy(acc.at[sl], out.at[c, sl])


_agg_kernel = pl.kernel(
    _agg_body,
    out_type=jax.ShapeDtypeStruct((NC, ACC_N, DHALF), jnp.float32),
    mesh=_mesh,
    scratch_types=[
        pltpu.VMEM((CH, B), jnp.int32),
        pltpu.VMEM((CH, B), jnp.int32),
        pltpu.VMEM((B, DHALF), jnp.float32),
        pltpu.VMEM((B, DHALF), jnp.float32),
        pltpu.VMEM_SHARED((ACC_N, DHALF), jnp.float32),
        pltpu.SemaphoreType.DMA,
        pltpu.SemaphoreType.DMA,
        pltpu.SemaphoreType.DMA,
        pltpu.SemaphoreType.DMA,
    ],
)


RB = 1000
GRID = N // RB


def _tc0_body(x_ref, w_ref, degp_ref, hlo_ref, hhi_ref):
    dinv = lax.rsqrt(1.0 + degp_ref[0, :, 0:1] + degp_ref[1, :, 0:1])
    h = jnp.dot(x_ref[...], w_ref[...], preferred_element_type=jnp.float32) * dinv
    hlo_ref[...] = h[:, :DHALF]
    hhi_ref[...] = h[:, DHALF:]


def _tcmid_body(s_ref, hlo_ref, hhi_ref, degp_ref, b_ref, w_ref, olo_ref, ohi_ref):
    dinv = lax.rsqrt(1.0 + degp_ref[0, :, 0:1] + degp_ref[1, :, 0:1])
    za = jnp.maximum(dinv * (s_ref[0] + hlo_ref[...]) + b_ref[0:1, :DHALF], 0.0)
    zb = jnp.maximum(dinv * (s_ref[1] + hhi_ref[...]) + b_ref[0:1, DHALF:], 0.0)
    z = jnp.concatenate([za, zb], axis=1)
    h = jnp.dot(z, w_ref[...], preferred_element_type=jnp.float32) * dinv
    olo_ref[...] = h[:, :DHALF]
    ohi_ref[...] = h[:, DHALF:]


def _tchead_body(s_ref, hlo_ref, hhi_ref, degp_ref, b_ref, wl_ref, bl_ref, out_ref):
    dinv = lax.rsqrt(1.0 + degp_ref[0, :, 0:1] + degp_ref[1, :, 0:1])
    za = jnp.maximum(dinv * (s_ref[0] + hlo_ref[...]) + b_ref[0:1, :DHALF], 0.0)
    zb = jnp.maximum(dinv * (s_ref[1] + hhi_ref[...]) + b_ref[0:1, DHALF:], 0.0)
    z = jnp.concatenate([za, zb], axis=1)
    out_ref[...] = (
        jnp.dot(z, wl_ref[...], preferred_element_type=jnp.float32) + bl_ref[0:1, :]
    )


def _row_spec(d):
    return pl.BlockSpec((RB, d), lambda i: (i, 0))


_SPEC_S = pl.BlockSpec((NC, RB, DHALF), lambda i: (0, i, 0))
_SPEC_DEG = pl.BlockSpec((NC, RB, DHALF), lambda i: (0, i, 0))


def _tc0(x, w0, degp):
    return pl.pallas_call(
        _tc0_body,
        grid=(GRID,),
        in_specs=[
            _row_spec(D_IN),
            pl.BlockSpec((D_IN, D_H), lambda i: (0, 0)),
            _SPEC_DEG,
        ],
        out_specs=[_row_spec(DHALF), _row_spec(DHALF)],
        out_shape=[
            jax.ShapeDtypeStruct((N, DHALF), jnp.float32),
            jax.ShapeDtypeStruct((N, DHALF), jnp.float32),
        ],
    )(x, w0, degp)


def _tcmid(s, hlo, hhi, degp, b2d, w):
    return pl.pallas_call(
        _tcmid_body,
        grid=(GRID,),
        in_specs=[
            _SPEC_S,
            _row_spec(DHALF),
            _row_spec(DHALF),
            _SPEC_DEG,
            pl.BlockSpec((1, D_H), lambda i: (0, 0)),
            pl.BlockSpec((D_H, D_H), lambda i: (0, 0)),
        ],
        out_specs=[_row_spec(DHALF), _row_spec(DHALF)],
        out_shape=[
            jax.ShapeDtypeStruct((N, DHALF), jnp.float32),
            jax.ShapeDtypeStruct((N, DHALF), jnp.float32),
        ],
    )(s, hlo, hhi, degp, b2d, w)


def _tchead(s, hlo, hhi, degp, b2d, wl, bl2d):
    return pl.pallas_call(
        _tchead_body,
        grid=(GRID,),
        in_specs=[
            _SPEC_S,
            _row_spec(DHALF),
            _row_spec(DHALF),
            _SPEC_DEG,
            pl.BlockSpec((1, D_H), lambda i: (0, 0)),
            pl.BlockSpec((D_H, D_OUT), lambda i: (0, 0)),
            pl.BlockSpec((1, D_OUT), lambda i: (0, 0)),
        ],
        out_specs=_row_spec(D_OUT),
        out_shape=jax.ShapeDtypeStruct((N, D_OUT), jnp.float32),
    )(s, hlo, hhi, degp, b2d, wl, bl2d)


def kernel(x, edge_index, W0, b0, W1, b1, W2, b2, Wl, bl):
    src = edge_index[0].astype(jnp.int32)
    dst = edge_index[1].astype(jnp.int32)
    pad = E_PAD - E
    src3 = jnp.concatenate([src, jnp.zeros((pad,), jnp.int32)]).reshape(
        NS, NCH, CH, B
    )
    dst3 = jnp.concatenate([dst, jnp.full((pad,), N, jnp.int32)]).reshape(
        NS, NCH, CH, B
    )

    zeros128 = jnp.zeros((ROWS_PER_TILE, DHALF), jnp.float32)
    ones128 = jnp.ones((B, DHALF), jnp.float32)

    degp = _deg_kernel(dst3, ones128, zeros128)

    hlo, hhi = _tc0(x, W0, degp)
    s = _agg_kernel(src3, dst3, hlo, hhi, zeros128)
    hlo, hhi = _tcmid(s, hlo, hhi, degp, b0.reshape(1, D_H), W1)
    s = _agg_kernel(src3, dst3, hlo, hhi, zeros128)
    hlo, hhi = _tcmid(s, hlo, hhi, degp, b1.reshape(1, D_H), W2)
    s = _agg_kernel(src3, dst3, hlo, hhi, zeros128)
    return _tchead(s, hlo, hhi, degp, b2.reshape(1, D_H), Wl, bl.reshape(1, D_OUT))
